# baseline (speedup 1.0000x reference)
import functools

import jax
import jax.numpy as jnp
from jax import lax
from jax.experimental import pallas as pl
from jax.experimental.pallas import tpu as pltpu
from jax.experimental.pallas import tpu_sc as plsc

NUM_USERS = 50000
NUM_ITEMS = 10000
DIM = 64
HALF = DIM // 2
TEMP = 0.2
BATCH = 1024
BEHAVIORS = ["pv", "cart", "buy"]
GRAPHS = ["pv", "cart", "buy", "all"]

NC, NS = 2, 16
CHUNK = 128
CB = 8
JUNK_U, JUNK_I = 176, 240
NU_P = NUM_USERS + JUNK_U
NI_P = NUM_ITEMS + JUNK_I
ZROWS = 224

E_PAD = {"pv": 212992, "cart": 212992, "buy": 212992, "all": 606208}

ITEM_BLK = 1024
N_IBLK = 10

_mesh = functools.partial(plsc.VectorSubcoreMesh, core_axis_name="c",
                          subcore_axis_name="s", num_cores=NC, num_subcores=NS)
_SC_PARAMS = pltpu.CompilerParams(use_tc_tiling_on_sc=False)


def _stripes(n):
    hi = ((n // NS) + 7) // 8 * 8
    lo = n - (NS - 1) * hi
    return hi, lo


def _copy_stripe(s, src, dst, n):
    hi, lo = _stripes(n)

    @pl.when(s < NS - 1)
    def _():
        pltpu.sync_copy(src.at[pl.ds(s * hi, hi)], dst.at[pl.ds(s * hi, hi)])

    @pl.when(s == NS - 1)
    def _():
        pltpu.sync_copy(src.at[pl.ds((NS - 1) * hi, lo)],
                        dst.at[pl.ds((NS - 1) * hi, lo)])


def _zero_acc(s, zbuf, acc, rows_pad, width_rows):
    per = rows_pad // NS
    base = s * per
    full, rem = per // ZROWS, per % ZROWS
    for k in range(full):
        pltpu.sync_copy(zbuf, acc.at[pl.ds(base + ZROWS * k, ZROWS)])
    if rem:
        pltpu.sync_copy(zbuf.at[pl.ds(0, rem)], acc.at[pl.ds(base + full * ZROWS, rem)])


def _deg_side(idx2d_list, acc, outs, s, idx_buf, obuf, sem_s, zbuf):
    for g, gname in enumerate(GRAPHS):
        _zero_acc(s, zbuf, acc, acc.shape[0], None)
        plsc.subcore_barrier()
        e2d = E_PAD[gname] // CHUNK
        rpt = e2d // NS
        nblk = rpt // CB
        row0 = s * rpt

        @pl.loop(0, nblk)
        def _blk(b, _row0=row0, _e=idx2d_list[g]):
            r = _row0 + b * CB
            pltpu.sync_copy(_e.at[pl.ds(r, CB)], idx_buf)
            ds = [pltpu.async_copy(obuf, acc.at[idx_buf.at[j]], sem_s, add=True)
                  for j in range(CB)]
            for d in ds:
                d.wait()

        plsc.subcore_barrier()
        _copy_stripe(s, acc, outs[g], outs[g].shape[0])
        plsc.subcore_barrier()


def _deg_body(refs):
    (eus, eis, z16, o16, degu_outs, degi_outs,
     acc_u, acc_i, zbuf, obuf, idx_buf, sem_s) = refs
    c = lax.axis_index("c")
    s = lax.axis_index("s")
    pltpu.sync_copy(z16, zbuf)
    pltpu.sync_copy(o16, obuf)

    @pl.when(c == 0)
    def _user_side():
        _deg_side(eus, acc_u, degu_outs, s, idx_buf, obuf, sem_s, zbuf)

    @pl.when(c == 1)
    def _item_side():
        _deg_side(eis, acc_i, degi_outs, s, idx_buf, obuf, sem_s, zbuf)


def _sc_degrees(eu2d, ei2d):
    z16 = jnp.zeros((ZROWS, 16), jnp.float32)
    o16 = jnp.ones((CHUNK, 16), jnp.float32)

    def body(*refs):
        _deg_body((refs[0:4], refs[4:8], refs[8], refs[9],
                   refs[10:14], refs[14:18],
                   refs[18], refs[19], refs[20], refs[21], refs[22], refs[23]))

    out_type = ([jax.ShapeDtypeStruct((NUM_USERS, 16), jnp.float32)] * 4
                + [jax.ShapeDtypeStruct((NUM_ITEMS, 16), jnp.float32)] * 4)
    f = pl.kernel(
        body, out_type=out_type, mesh=_mesh(), compiler_params=_SC_PARAMS,
        scratch_types=[
            pltpu.VMEM_SHARED((NU_P, 16), jnp.float32),
            pltpu.VMEM_SHARED((NI_P, 16), jnp.float32),
            pltpu.VMEM((ZROWS, 16), jnp.float32),
            pltpu.VMEM((CHUNK, 16), jnp.float32),
            pltpu.VMEM((CB, CHUNK), jnp.int32),
            pltpu.SemaphoreType.DMA,
        ])
    outs = f(*eu2d, *ei2d, z16, o16)
    degu = [outs[g][:, 0] for g in range(4)]
    degi = [outs[4 + g][:, 0] for g in range(4)]
    return degu, degi, list(outs[0:4]), list(outs[4:8])


def _pass_body(refs, n_acc, rcb):
    (tabs, srcs, dsts, z32, outs,
     acc, zbuf, idx_s, idx_d, idx_s2, idx_d2, rows, sem_g, sem_s, sem_i) = refs
    c = lax.axis_index("c")
    s = lax.axis_index("s")
    pltpu.sync_copy(z32, zbuf)
    for g, gname in enumerate(GRAPHS):
        _zero_acc(s, zbuf, acc, n_acc, None)
        plsc.subcore_barrier()

        e2d = E_PAD[gname] // CHUNK
        rpt = e2d // NS
        nblk = rpt // CB
        row0 = s * rpt

        nsub = CB // rcb

        def _do_block(idx_s, idx_d, _tab):
            gd, sd = {}, {}
            gd[0] = [pltpu.async_copy(_tab.at[c].at[idx_s.at[j]],
                                      rows.at[0].at[j], sem_g)
                     for j in range(rcb)]
            for k in range(nsub):
                if k + 1 < nsub:
                    if k >= 1:
                        for d in sd[k - 1]:
                            d.wait()
                    nb = (k + 1) % 2
                    gd[k + 1] = [pltpu.async_copy(
                        _tab.at[c].at[idx_s.at[(k + 1) * rcb + j]],
                        rows.at[nb].at[j], sem_g) for j in range(rcb)]
                for d in gd[k]:
                    d.wait()
                sd[k] = [pltpu.async_copy(rows.at[k % 2].at[j],
                                          acc.at[idx_d.at[k * rcb + j]],
                                          sem_s, add=True)
                         for j in range(rcb)]
            for k in range(max(0, nsub - 2), nsub):
                for d in sd[k]:
                    d.wait()

        nblk2, blk_tail = nblk // 2, nblk % 2

        @pl.loop(0, nblk2)
        def _blk(t, _row0=row0, _src=srcs[g], _dst=dsts[g], _tab=tabs[g]):
            r = _row0 + t * 2 * CB
            pltpu.sync_copy(_src.at[pl.ds(r, CB)], idx_s)
            pltpu.sync_copy(_dst.at[pl.ds(r, CB)], idx_d)
            pf = [pltpu.async_copy(_src.at[pl.ds(r + CB, CB)], idx_s2, sem_i),
                  pltpu.async_copy(_dst.at[pl.ds(r + CB, CB)], idx_d2, sem_i)]
            _do_block(idx_s, idx_d, _tab)
            for d in pf:
                d.wait()
            _do_block(idx_s2, idx_d2, _tab)

        if blk_tail:
            rt = row0 + (nblk - 1) * CB
            pltpu.sync_copy(srcs[g].at[pl.ds(rt, CB)], idx_s)
            pltpu.sync_copy(dsts[g].at[pl.ds(rt, CB)], idx_d)
            _do_block(idx_s, idx_d, tabs[g])

        plsc.subcore_barrier()
        per = n_acc // NS
        pltpu.sync_copy(acc.at[pl.ds(s * per, per)],
                        outs[g].at[c].at[pl.ds(s * per, per)])
        plsc.subcore_barrier()


def _sc_pass(tables, srcs, dsts, n_acc, rcb):
    z32 = jnp.zeros((ZROWS, HALF), jnp.float32)

    def body(*refs):
        _pass_body((refs[0:4], refs[4:8], refs[8:12], refs[12],
                    refs[13:17],
                    refs[17], refs[18], refs[19], refs[20], refs[21],
                    refs[22], refs[23], refs[24], refs[25], refs[26]),
                   n_acc, rcb)

    out_type = [jax.ShapeDtypeStruct((NC, n_acc, HALF), jnp.float32)] * 4
    f = pl.kernel(
        body, out_type=out_type, mesh=_mesh(), compiler_params=_SC_PARAMS,
        scratch_types=[
            pltpu.VMEM_SHARED((n_acc, HALF), jnp.float32),
            pltpu.VMEM((ZROWS, HALF), jnp.float32),
            pltpu.VMEM((CB, CHUNK), jnp.int32),
            pltpu.VMEM((CB, CHUNK), jnp.int32),
            pltpu.VMEM((CB, CHUNK), jnp.int32),
            pltpu.VMEM((CB, CHUNK), jnp.int32),
            pltpu.VMEM((2, rcb, CHUNK, HALF), jnp.float32),
            pltpu.SemaphoreType.DMA,
            pltpu.SemaphoreType.DMA,
            pltpu.SemaphoreType.DMA,
        ])
    return list(f(*tables, *srcs, *dsts, z32))


def _sc_stage(tu, ti, eu2d, ei2d):
    item_accs = _sc_pass(tu, eu2d, ei2d, NI_P, 4)
    user_accs = _sc_pass(ti, ei2d, eu2d, NU_P, 2)
    return user_accs, item_accs


def _gather_body(refs):
    (qs, ts, degs, u0_in, scale_in, bidx, qouts, touts, dout, u0out, sout,
     idx64, idxw, rows32, rows64, rows16, sem_g) = refs
    c = lax.axis_index("c")
    s = lax.axis_index("s")
    w = s * NC + c
    pw = BATCH // (NC * NS)
    per = BATCH // NS
    pltpu.sync_copy(bidx.at[pl.ds(s * per, per)], idx64)
    for g in range(4):
        pltpu.async_copy(qs[g].at[c].at[idx64], rows32, sem_g).wait()
        pltpu.sync_copy(rows32, qouts[g].at[c].at[pl.ds(s * per, per)])
        pltpu.async_copy(ts[g].at[c].at[idx64], rows32, sem_g).wait()
        pltpu.sync_copy(rows32, touts[g].at[c].at[pl.ds(s * per, per)])
    pltpu.sync_copy(bidx.at[pl.ds(w * pw, pw)], idxw)
    for g in range(4):
        pltpu.async_copy(degs[g].at[idxw], rows16, sem_g).wait()
        pltpu.sync_copy(rows16, dout.at[g].at[pl.ds(w * pw, pw)])
    pltpu.async_copy(u0_in.at[idxw], rows64, sem_g).wait()
    pltpu.sync_copy(rows64, u0out.at[pl.ds(w * pw, pw)])
    pltpu.async_copy(scale_in.at[idxw], rows16, sem_g).wait()
    pltpu.sync_copy(rows16, sout.at[pl.ds(w * pw, pw)])


def _sc_gather(Q, T, deg16, u0, scale_user, batch_users):
    def body(*refs):
        _gather_body((refs[0:4], refs[4:8], refs[8:12], refs[12], refs[13],
                      refs[14],
                      refs[15:19], refs[19:23], refs[23], refs[24], refs[25],
                      refs[26], refs[27], refs[28], refs[29], refs[30],
                      refs[31]))

    pw = BATCH // (NC * NS)
    out_type = ([jax.ShapeDtypeStruct((NC, BATCH, HALF), jnp.float32)] * 8
                + [jax.ShapeDtypeStruct((4, BATCH, 16), jnp.float32)]
                + [jax.ShapeDtypeStruct((BATCH, DIM), jnp.float32)]
                + [jax.ShapeDtypeStruct((BATCH, 16), jnp.float32)])
    f = pl.kernel(
        body, out_type=out_type, mesh=_mesh(), compiler_params=_SC_PARAMS,
        scratch_types=[
            pltpu.VMEM((BATCH // NS,), jnp.int32),
            pltpu.VMEM((pw,), jnp.int32),
            pltpu.VMEM((BATCH // NS, HALF), jnp.float32),
            pltpu.VMEM((pw, DIM), jnp.float32),
            pltpu.VMEM((pw, 16), jnp.float32),
            pltpu.SemaphoreType.DMA,
        ])
    scale16 = jnp.pad(scale_user, ((0, 0), (0, 12)))
    outs = f(*Q, *T, *deg16, u0, scale16, batch_users)
    qb = [jnp.concatenate([o[0], o[1]], axis=1) for o in outs[0:4]]
    tb = [jnp.concatenate([o[0], o[1]], axis=1) for o in outs[4:8]]
    ru_b = jax.lax.rsqrt(jnp.maximum(outs[8][:, :, 0:1], 1.0))
    return qb, tb, ru_b, outs[9], outs[10][:, :4]


def _normalize(x):
    return x / (jnp.sqrt(jnp.sum(x * x, axis=-1, keepdims=True)) + 1e-12)


def _tail_kernel(u_pv, u_cart, u_buy, u_all,
                 p_pv, p_cart, p_buy, p_all,
                 r_pv, r_cart, r_buy, r_all,
                 i0_ref, d_pv, d_cart, d_buy, d_all,
                 gw_u, gb_u, gw_i, gb_i, w_beh, beh,
                 ssl_ref, likes_ref):
    blk = pl.program_id(0)

    @pl.when(blk == 0)
    def _ssl():
        z_buy = _normalize(u_buy[...])
        ssl = jnp.zeros((), jnp.float32)
        for z1 in (_normalize(u_pv[...]), _normalize(u_cart[...])):
            logits = jax.lax.dot_general(
                z1, z_buy, (((1,), (1,)), ((), ())),
                preferred_element_type=jnp.float32) / TEMP
            m = jnp.max(logits, axis=1, keepdims=True)
            lse = jnp.log(jnp.sum(jnp.exp(logits - m), axis=1, keepdims=True)) + m
            diag = jnp.sum(z1 * z_buy, axis=1, keepdims=True) / TEMP
            ssl = ssl + jnp.mean(lse - diag)
        ssl_ref[...] = ssl[None, None]

    col = jax.lax.broadcasted_iota(jnp.int32, (1, ITEM_BLK), 1) + blk * ITEM_BLK
    keep = (col != NUM_ITEMS - 1) & (col < NUM_ITEMS)

    users = {"pv": u_pv, "cart": u_cart, "buy": u_buy}
    i0v = i0_ref[...]
    icat = {}
    for k, pref, rref, dref in (("pv", p_pv, r_pv, d_pv),
                                ("cart", p_cart, r_cart, d_cart),
                                ("buy", p_buy, r_buy, d_buy),
                                ("all", p_all, r_all, d_all)):
        pr = pref[...] + rref[...]
        prc = jnp.concatenate([pr[0], pr[1]], axis=1)
        riv = jax.lax.rsqrt(jnp.maximum(dref[...][:, 0:1], 1.0))
        icat[k] = (i0v + riv * prc) / 3.0
    i_all_c = icat["all"]
    for b, name in enumerate(BEHAVIORS):
        e1u = u_all[...] + beh[b, :][None, :]
        e2u = users[name][...]
        gu = jax.nn.sigmoid(
            jax.lax.dot_general(e1u, gw_u[b], (((1,), (1,)), ((), ())),
                                preferred_element_type=jnp.float32) + gb_u[b, :][None, :])
        ub = gu * e1u + (1.0 - gu) * e2u
        ub = _normalize(ub) * w_beh[b, :][None, :]

        e1i = i_all_c
        e2i = icat[name]
        gi = jax.nn.sigmoid(
            jax.lax.dot_general(e1i, gw_i[b], (((1,), (1,)), ((), ())),
                                preferred_element_type=jnp.float32) + gb_i[b, :][None, :])
        ib = gi * e1i + (1.0 - gi) * e2i

        like = jax.lax.dot_general(ub, ib, (((1,), (1,)), ((), ())),
                                   preferred_element_type=jnp.float32)
        likes_ref[b, :, :] = jnp.where(keep, jax.nn.relu(like), 0.0)


def _dense_tail(embs_u, P, R, i0, degi16, params):
    gw_u = jnp.stack([params["gate_user_%s_W" % b] for b in BEHAVIORS])
    gb_u = jnp.stack([params["gate_user_%s_b" % b] for b in BEHAVIORS])
    gw_i = jnp.stack([params["gate_item_%s_W" % b] for b in BEHAVIORS])
    gb_i = jnp.stack([params["gate_item_%s_b" % b] for b in BEHAVIORS])
    w_beh = jnp.stack([params["w_" + b][:, 0] for b in BEHAVIORS])
    beh = jnp.stack([params["beh_" + b] for b in BEHAVIORS])

    u_spec = pl.BlockSpec((BATCH, DIM), lambda i: (0, 0))
    i_spec = pl.BlockSpec((NC, ITEM_BLK, HALF), lambda i: (0, i, 0))
    i0_spec = pl.BlockSpec((ITEM_BLK, DIM), lambda i: (i, 0))
    d_spec = pl.BlockSpec((ITEM_BLK, 16), lambda i: (i, 0))
    w_spec3 = pl.BlockSpec((3, DIM, DIM), lambda i: (0, 0, 0))
    v_spec3 = pl.BlockSpec((3, DIM), lambda i: (0, 0))

    ssl, likes = pl.pallas_call(
        _tail_kernel,
        grid=(N_IBLK,),
        in_specs=([u_spec] * 4 + [i_spec] * 8 + [i0_spec] + [d_spec] * 4
                  + [w_spec3, v_spec3, w_spec3, v_spec3, v_spec3, v_spec3]),
        out_specs=[
            pl.BlockSpec((1, 1), lambda i: (0, 0)),
            pl.BlockSpec((3, BATCH, ITEM_BLK), lambda i: (0, 0, i)),
        ],
        out_shape=[
            jax.ShapeDtypeStruct((1, 1), jnp.float32),
            jax.ShapeDtypeStruct((3, BATCH, NUM_ITEMS), jnp.float32),
        ],
    )(embs_u["pv"], embs_u["cart"], embs_u["buy"], embs_u["all"],
      P[0], P[1], P[2], P[3], R[0], R[1], R[2], R[3], i0, degi16[0],
      degi16[1], degi16[2], degi16[3],
      gw_u, gb_u, gw_i, gb_i, w_beh, beh)
    return ssl.reshape(()), likes


def _pad_edges(e, n_pad, junk_base, junk_n):
    k = jnp.arange(n_pad - e.shape[0], dtype=jnp.int32)
    return jnp.concatenate([e, junk_base + (k % junk_n)]).reshape(-1, CHUNK)


def _halves_pad(x, rows_pad):
    h = jnp.stack([x[:, :HALF], x[:, HALF:]])
    return jnp.pad(h, ((0, 0), (0, rows_pad - x.shape[0]), (0, 0)))


def kernel(batch_users, whole_items, dropout_ration, params, edges):
    u0, i0 = params["user_table"], params["item_table"]

    eu2d = [_pad_edges(edges[g + "_u"], E_PAD[g], NUM_USERS, JUNK_U) for g in GRAPHS]
    ei2d = [_pad_edges(edges[g + "_i"], E_PAD[g], NUM_ITEMS, JUNK_I) for g in GRAPHS]

    degu, degi, degu16, degi16 = _sc_degrees(eu2d, ei2d)
    u0h = _halves_pad(u0, NU_P)
    i0h = _halves_pad(i0, NI_P)
    ru = [jnp.pad(jax.lax.rsqrt(jnp.maximum(degu[g], 1.0)), (0, JUNK_U))[None, :, None]
          for g in range(4)]
    ri = [jnp.pad(jax.lax.rsqrt(jnp.maximum(degi[g], 1.0)), (0, JUNK_I))[None, :, None]
          for g in range(4)]

    Q, P = _sc_stage([u0h * r for r in ru], [i0h * r for r in ri], eu2d, ei2d)
    T, R = _sc_stage([Q[g] * (ru[g] * ru[g]) for g in range(4)],
                     [P[g] * (ri[g] * ri[g]) for g in range(4)], eu2d, ei2d)

    qb, tb, ru_b, u0b, scale = _sc_gather(Q, T, degu16, u0,
                                          params["scale_user"], batch_users)
    embs_u = {g: (u0b + ru_b[gi] * (qb[gi] + tb[gi])) / 3.0
              for gi, g in enumerate(GRAPHS)}
    ssl, likes = _dense_tail(embs_u, P, R, i0, degi16, params)
    return (ssl, likes, scale)


# speedup vs baseline: 1.0427x; 1.0427x over previous
import functools

import jax
import jax.numpy as jnp
from jax import lax
from jax.experimental import pallas as pl
from jax.experimental.pallas import tpu as pltpu
from jax.experimental.pallas import tpu_sc as plsc

NUM_USERS = 50000
NUM_ITEMS = 10000
DIM = 64
HALF = DIM // 2
TEMP = 0.2
BATCH = 1024
BEHAVIORS = ["pv", "cart", "buy"]
GRAPHS = ["pv", "cart", "buy", "all"]

NC, NS = 2, 16
CHUNK = 128
CB = 8
JUNK_U, JUNK_I = 176, 240
NU_P = NUM_USERS + JUNK_U
NI_P = NUM_ITEMS + JUNK_I
ZROWS = 224

E_PAD = {"pv": 212992, "cart": 212992, "buy": 212992, "all": 606208}

ITEM_BLK = 1024
N_IBLK = 10

_mesh = functools.partial(plsc.VectorSubcoreMesh, core_axis_name="c",
                          subcore_axis_name="s", num_cores=NC, num_subcores=NS)
_SC_PARAMS = pltpu.CompilerParams(use_tc_tiling_on_sc=False)


def _stripes(n):
    hi = ((n // NS) + 7) // 8 * 8
    lo = n - (NS - 1) * hi
    return hi, lo


def _copy_stripe(s, src, dst, n):
    hi, lo = _stripes(n)

    @pl.when(s < NS - 1)
    def _():
        pltpu.sync_copy(src.at[pl.ds(s * hi, hi)], dst.at[pl.ds(s * hi, hi)])

    @pl.when(s == NS - 1)
    def _():
        pltpu.sync_copy(src.at[pl.ds((NS - 1) * hi, lo)],
                        dst.at[pl.ds((NS - 1) * hi, lo)])


def _zero_acc(s, zbuf, acc, rows_pad, width_rows):
    per = rows_pad // NS
    base = s * per
    full, rem = per // ZROWS, per % ZROWS
    for k in range(full):
        pltpu.sync_copy(zbuf, acc.at[pl.ds(base + ZROWS * k, ZROWS)])
    if rem:
        pltpu.sync_copy(zbuf.at[pl.ds(0, rem)], acc.at[pl.ds(base + full * ZROWS, rem)])


def _deg_side(idx2d_list, acc, outs, s, idx_buf, obuf, sem_s, zbuf):
    for g, gname in enumerate(GRAPHS):
        _zero_acc(s, zbuf, acc, acc.shape[0], None)
        plsc.subcore_barrier()
        e2d = E_PAD[gname] // CHUNK
        rpt = e2d // NS
        nblk = rpt // CB
        row0 = s * rpt

        @pl.loop(0, nblk)
        def _blk(b, _row0=row0, _e=idx2d_list[g]):
            r = _row0 + b * CB
            pltpu.sync_copy(_e.at[pl.ds(r, CB)], idx_buf)
            ds = [pltpu.async_copy(obuf, acc.at[idx_buf.at[j]], sem_s, add=True)
                  for j in range(CB)]
            for d in ds:
                d.wait()

        plsc.subcore_barrier()
        _copy_stripe(s, acc, outs[g], outs[g].shape[0])
        plsc.subcore_barrier()


def _deg_body(refs):
    (eus, eis, z16, o16, degu_outs, degi_outs,
     acc_u, acc_i, zbuf, obuf, idx_buf, sem_s) = refs
    c = lax.axis_index("c")
    s = lax.axis_index("s")
    pltpu.sync_copy(z16, zbuf)
    pltpu.sync_copy(o16, obuf)

    @pl.when(c == 0)
    def _user_side():
        _deg_side(eus, acc_u, degu_outs, s, idx_buf, obuf, sem_s, zbuf)

    @pl.when(c == 1)
    def _item_side():
        _deg_side(eis, acc_i, degi_outs, s, idx_buf, obuf, sem_s, zbuf)


def _sc_degrees(eu2d, ei2d):
    z16 = jnp.zeros((ZROWS, 16), jnp.float32)
    o16 = jnp.ones((CHUNK, 16), jnp.float32)

    def body(*refs):
        _deg_body((refs[0:4], refs[4:8], refs[8], refs[9],
                   refs[10:14], refs[14:18],
                   refs[18], refs[19], refs[20], refs[21], refs[22], refs[23]))

    out_type = ([jax.ShapeDtypeStruct((NUM_USERS, 16), jnp.float32)] * 4
                + [jax.ShapeDtypeStruct((NUM_ITEMS, 16), jnp.float32)] * 4)
    f = pl.kernel(
        body, out_type=out_type, mesh=_mesh(), compiler_params=_SC_PARAMS,
        scratch_types=[
            pltpu.VMEM_SHARED((NU_P, 16), jnp.float32),
            pltpu.VMEM_SHARED((NI_P, 16), jnp.float32),
            pltpu.VMEM((ZROWS, 16), jnp.float32),
            pltpu.VMEM((CHUNK, 16), jnp.float32),
            pltpu.VMEM((CB, CHUNK), jnp.int32),
            pltpu.SemaphoreType.DMA,
        ])
    outs = f(*eu2d, *ei2d, z16, o16)
    degu = [outs[g][:, 0] for g in range(4)]
    degi = [outs[4 + g][:, 0] for g in range(4)]
    return degu, degi, list(outs[0:4])


def _pass_body(refs, n_acc, rcb):
    (tabs, srcs, dsts, z32, outs,
     acc, zbuf, idx_s, idx_d, idx_s2, idx_d2, rows, sem_g, sem_s, sem_i) = refs
    c = lax.axis_index("c")
    s = lax.axis_index("s")
    pltpu.sync_copy(z32, zbuf)
    for g, gname in enumerate(GRAPHS):
        _zero_acc(s, zbuf, acc, n_acc, None)
        plsc.subcore_barrier()

        e2d = E_PAD[gname] // CHUNK
        rpt = e2d // NS
        nblk = rpt // CB
        row0 = s * rpt

        nsub = CB // rcb

        def _do_block(idx_s, idx_d, _tab):
            gd, sd = {}, {}
            gd[0] = [pltpu.async_copy(_tab.at[c].at[idx_s.at[j]],
                                      rows.at[0].at[j], sem_g)
                     for j in range(rcb)]
            for k in range(nsub):
                if k + 1 < nsub:
                    if k >= 1:
                        for d in sd[k - 1]:
                            d.wait()
                    nb = (k + 1) % 2
                    gd[k + 1] = [pltpu.async_copy(
                        _tab.at[c].at[idx_s.at[(k + 1) * rcb + j]],
                        rows.at[nb].at[j], sem_g) for j in range(rcb)]
                for d in gd[k]:
                    d.wait()
                sd[k] = [pltpu.async_copy(rows.at[k % 2].at[j],
                                          acc.at[idx_d.at[k * rcb + j]],
                                          sem_s, add=True)
                         for j in range(rcb)]
            for k in range(max(0, nsub - 2), nsub):
                for d in sd[k]:
                    d.wait()

        nblk2, blk_tail = nblk // 2, nblk % 2

        @pl.loop(0, nblk2)
        def _blk(t, _row0=row0, _src=srcs[g], _dst=dsts[g], _tab=tabs[g]):
            r = _row0 + t * 2 * CB
            pltpu.sync_copy(_src.at[pl.ds(r, CB)], idx_s)
            pltpu.sync_copy(_dst.at[pl.ds(r, CB)], idx_d)
            pf = [pltpu.async_copy(_src.at[pl.ds(r + CB, CB)], idx_s2, sem_i),
                  pltpu.async_copy(_dst.at[pl.ds(r + CB, CB)], idx_d2, sem_i)]
            _do_block(idx_s, idx_d, _tab)
            for d in pf:
                d.wait()
            _do_block(idx_s2, idx_d2, _tab)

        if blk_tail:
            rt = row0 + (nblk - 1) * CB
            pltpu.sync_copy(srcs[g].at[pl.ds(rt, CB)], idx_s)
            pltpu.sync_copy(dsts[g].at[pl.ds(rt, CB)], idx_d)
            _do_block(idx_s, idx_d, tabs[g])

        plsc.subcore_barrier()
        per = n_acc // NS
        pltpu.sync_copy(acc.at[pl.ds(s * per, per)],
                        outs[g].at[c].at[pl.ds(s * per, per)])
        plsc.subcore_barrier()


def _sc_pass(tables, srcs, dsts, n_acc, rcb):
    z32 = jnp.zeros((ZROWS, HALF), jnp.float32)

    def body(*refs):
        _pass_body((refs[0:4], refs[4:8], refs[8:12], refs[12],
                    refs[13:17],
                    refs[17], refs[18], refs[19], refs[20], refs[21],
                    refs[22], refs[23], refs[24], refs[25], refs[26]),
                   n_acc, rcb)

    out_type = [jax.ShapeDtypeStruct((NC, n_acc, HALF), jnp.float32)] * 4
    f = pl.kernel(
        body, out_type=out_type, mesh=_mesh(), compiler_params=_SC_PARAMS,
        scratch_types=[
            pltpu.VMEM_SHARED((n_acc, HALF), jnp.float32),
            pltpu.VMEM((ZROWS, HALF), jnp.float32),
            pltpu.VMEM((CB, CHUNK), jnp.int32),
            pltpu.VMEM((CB, CHUNK), jnp.int32),
            pltpu.VMEM((CB, CHUNK), jnp.int32),
            pltpu.VMEM((CB, CHUNK), jnp.int32),
            pltpu.VMEM((2, rcb, CHUNK, HALF), jnp.float32),
            pltpu.SemaphoreType.DMA,
            pltpu.SemaphoreType.DMA,
            pltpu.SemaphoreType.DMA,
        ])
    return list(f(*tables, *srcs, *dsts, z32))


def _sc_stage(tu, ti, eu2d, ei2d):
    item_accs = _sc_pass(tu, eu2d, ei2d, NI_P, 4)
    user_accs = _sc_pass(ti, ei2d, eu2d, NU_P, 2)
    return user_accs, item_accs


def _gather_body(refs):
    (qs, ts, degs, u0_in, scale_in, bidx, qouts, touts, dout, u0out, sout,
     idx64, idxw, rows32, rows64, rows16, sem_g) = refs
    c = lax.axis_index("c")
    s = lax.axis_index("s")
    w = s * NC + c
    pw = BATCH // (NC * NS)
    per = BATCH // NS
    pltpu.sync_copy(bidx.at[pl.ds(s * per, per)], idx64)
    for g in range(4):
        pltpu.async_copy(qs[g].at[c].at[idx64], rows32, sem_g).wait()
        pltpu.sync_copy(rows32, qouts[g].at[c].at[pl.ds(s * per, per)])
        pltpu.async_copy(ts[g].at[c].at[idx64], rows32, sem_g).wait()
        pltpu.sync_copy(rows32, touts[g].at[c].at[pl.ds(s * per, per)])
    pltpu.sync_copy(bidx.at[pl.ds(w * pw, pw)], idxw)
    for g in range(4):
        pltpu.async_copy(degs[g].at[idxw], rows16, sem_g).wait()
        pltpu.sync_copy(rows16, dout.at[g].at[pl.ds(w * pw, pw)])
    pltpu.async_copy(u0_in.at[idxw], rows64, sem_g).wait()
    pltpu.sync_copy(rows64, u0out.at[pl.ds(w * pw, pw)])
    pltpu.async_copy(scale_in.at[idxw], rows16, sem_g).wait()
    pltpu.sync_copy(rows16, sout.at[pl.ds(w * pw, pw)])


def _sc_gather(Q, T, deg16, u0, scale_user, batch_users):
    def body(*refs):
        _gather_body((refs[0:4], refs[4:8], refs[8:12], refs[12], refs[13],
                      refs[14],
                      refs[15:19], refs[19:23], refs[23], refs[24], refs[25],
                      refs[26], refs[27], refs[28], refs[29], refs[30],
                      refs[31]))

    pw = BATCH // (NC * NS)
    out_type = ([jax.ShapeDtypeStruct((NC, BATCH, HALF), jnp.float32)] * 8
                + [jax.ShapeDtypeStruct((4, BATCH, 16), jnp.float32)]
                + [jax.ShapeDtypeStruct((BATCH, DIM), jnp.float32)]
                + [jax.ShapeDtypeStruct((BATCH, 16), jnp.float32)])
    f = pl.kernel(
        body, out_type=out_type, mesh=_mesh(), compiler_params=_SC_PARAMS,
        scratch_types=[
            pltpu.VMEM((BATCH // NS,), jnp.int32),
            pltpu.VMEM((pw,), jnp.int32),
            pltpu.VMEM((BATCH // NS, HALF), jnp.float32),
            pltpu.VMEM((pw, DIM), jnp.float32),
            pltpu.VMEM((pw, 16), jnp.float32),
            pltpu.SemaphoreType.DMA,
        ])
    scale16 = jnp.pad(scale_user, ((0, 0), (0, 12)))
    outs = f(*Q, *T, *deg16, u0, scale16, batch_users)
    qb = [jnp.concatenate([o[0], o[1]], axis=1) for o in outs[0:4]]
    tb = [jnp.concatenate([o[0], o[1]], axis=1) for o in outs[4:8]]
    ru_b = jax.lax.rsqrt(jnp.maximum(outs[8][:, :, 0:1], 1.0))
    return qb, tb, ru_b, outs[9], outs[10][:, :4]


def _normalize(x):
    return x / (jnp.sqrt(jnp.sum(x * x, axis=-1, keepdims=True)) + 1e-12)


def _tail_kernel(u_pv, u_cart, u_buy, u_all,
                 i_pv, i_cart, i_buy, i_all,
                 gw_u, gb_u, gw_i, gb_i, w_beh, beh,
                 ssl_ref, likes_ref):
    blk = pl.program_id(0)

    @pl.when(blk == 0)
    def _ssl():
        z_buy = _normalize(u_buy[...])
        ssl = jnp.zeros((), jnp.float32)
        for z1 in (_normalize(u_pv[...]), _normalize(u_cart[...])):
            logits = jax.lax.dot_general(
                z1, z_buy, (((1,), (1,)), ((), ())),
                preferred_element_type=jnp.float32) / TEMP
            m = jnp.max(logits, axis=1, keepdims=True)
            lse = jnp.log(jnp.sum(jnp.exp(logits - m), axis=1, keepdims=True)) + m
            diag = jnp.sum(z1 * z_buy, axis=1, keepdims=True) / TEMP
            ssl = ssl + jnp.mean(lse - diag)
        ssl_ref[...] = ssl[None, None]

    col = jax.lax.broadcasted_iota(jnp.int32, (1, ITEM_BLK), 1) + blk * ITEM_BLK
    keep = (col != NUM_ITEMS - 1) & (col < NUM_ITEMS)

    users = {"pv": u_pv, "cart": u_cart, "buy": u_buy}
    items = {"pv": i_pv, "cart": i_cart, "buy": i_buy}
    icat = {k: jnp.concatenate([v[0], v[1]], axis=1) / 3.0
            for k, v in items.items()}
    i_all_c = jnp.concatenate([i_all[0], i_all[1]], axis=1) / 3.0
    for b, name in enumerate(BEHAVIORS):
        e1u = u_all[...] + beh[b, :][None, :]
        e2u = users[name][...]
        gu = jax.nn.sigmoid(
            jax.lax.dot_general(e1u, gw_u[b], (((1,), (1,)), ((), ())),
                                preferred_element_type=jnp.float32) + gb_u[b, :][None, :])
        ub = gu * e1u + (1.0 - gu) * e2u
        ub = _normalize(ub) * w_beh[b, :][None, :]

        e1i = i_all_c
        e2i = icat[name]
        gi = jax.nn.sigmoid(
            jax.lax.dot_general(e1i, gw_i[b], (((1,), (1,)), ((), ())),
                                preferred_element_type=jnp.float32) + gb_i[b, :][None, :])
        ib = gi * e1i + (1.0 - gi) * e2i

        like = jax.lax.dot_general(ub, ib, (((1,), (1,)), ((), ())),
                                   preferred_element_type=jnp.float32)
        likes_ref[b, :, :] = jnp.where(keep, jax.nn.relu(like), 0.0)


def _dense_tail(embs_u, embs_i, params):
    gw_u = jnp.stack([params["gate_user_%s_W" % b] for b in BEHAVIORS])
    gb_u = jnp.stack([params["gate_user_%s_b" % b] for b in BEHAVIORS])
    gw_i = jnp.stack([params["gate_item_%s_W" % b] for b in BEHAVIORS])
    gb_i = jnp.stack([params["gate_item_%s_b" % b] for b in BEHAVIORS])
    w_beh = jnp.stack([params["w_" + b][:, 0] for b in BEHAVIORS])
    beh = jnp.stack([params["beh_" + b] for b in BEHAVIORS])

    u_spec = pl.BlockSpec((BATCH, DIM), lambda i: (0, 0))
    i_spec = pl.BlockSpec((NC, ITEM_BLK, HALF), lambda i: (0, i, 0))
    w_spec3 = pl.BlockSpec((3, DIM, DIM), lambda i: (0, 0, 0))
    v_spec3 = pl.BlockSpec((3, DIM), lambda i: (0, 0))

    ssl, likes = pl.pallas_call(
        _tail_kernel,
        grid=(N_IBLK,),
        in_specs=[u_spec] * 4 + [i_spec] * 4 + [w_spec3, v_spec3, w_spec3, v_spec3, v_spec3, v_spec3],
        out_specs=[
            pl.BlockSpec((1, 1), lambda i: (0, 0)),
            pl.BlockSpec((3, BATCH, ITEM_BLK), lambda i: (0, 0, i)),
        ],
        out_shape=[
            jax.ShapeDtypeStruct((1, 1), jnp.float32),
            jax.ShapeDtypeStruct((3, BATCH, NUM_ITEMS), jnp.float32),
        ],
    )(embs_u["pv"], embs_u["cart"], embs_u["buy"], embs_u["all"],
      embs_i[0], embs_i[1], embs_i[2], embs_i[3],
      gw_u, gb_u, gw_i, gb_i, w_beh, beh)
    return ssl.reshape(()), likes


def _pad_edges(e, n_pad, junk_base, junk_n):
    k = jnp.arange(n_pad - e.shape[0], dtype=jnp.int32)
    return jnp.concatenate([e, junk_base + (k % junk_n)]).reshape(-1, CHUNK)


def _halves_pad(x, rows_pad):
    h = jnp.stack([x[:, :HALF], x[:, HALF:]])
    return jnp.pad(h, ((0, 0), (0, rows_pad - x.shape[0]), (0, 0)))


def kernel(batch_users, whole_items, dropout_ration, params, edges):
    u0, i0 = params["user_table"], params["item_table"]

    eu2d = [_pad_edges(edges[g + "_u"], E_PAD[g], NUM_USERS, JUNK_U) for g in GRAPHS]
    ei2d = [_pad_edges(edges[g + "_i"], E_PAD[g], NUM_ITEMS, JUNK_I) for g in GRAPHS]

    degu, degi, degu16 = _sc_degrees(eu2d, ei2d)
    u0h = _halves_pad(u0, NU_P)
    i0h = _halves_pad(i0, NI_P)
    ru = [jnp.pad(jax.lax.rsqrt(jnp.maximum(degu[g], 1.0)), (0, JUNK_U))[None, :, None]
          for g in range(4)]
    ri = [jnp.pad(jax.lax.rsqrt(jnp.maximum(degi[g], 1.0)), (0, JUNK_I))[None, :, None]
          for g in range(4)]

    Q, P = _sc_stage([u0h * r for r in ru], [i0h * r for r in ri], eu2d, ei2d)
    T, R = _sc_stage([Q[g] * (ru[g] * ru[g]) for g in range(4)],
                     [P[g] * (ri[g] * ri[g]) for g in range(4)], eu2d, ei2d)

    iacc = [i0h + ri[g] * (P[g] + R[g]) for g in range(4)]

    qb, tb, ru_b, u0b, scale = _sc_gather(Q, T, degu16, u0,
                                          params["scale_user"], batch_users)
    embs_u = {g: (u0b + ru_b[gi] * (qb[gi] + tb[gi])) / 3.0
              for gi, g in enumerate(GRAPHS)}
    ssl, likes = _dense_tail(embs_u, iacc, params)
    return (ssl, likes, scale)
